# initial kernel scaffold (unmeasured)
import jax
import jax.numpy as jnp
from jax import lax
from jax.experimental import pallas as pl
from jax.experimental.pallas import tpu as pltpu


def kernel(
    x,
):
    def body(*refs):
        pass

    out_shape = jax.ShapeDtypeStruct(..., jnp.float32)
    return pl.pallas_call(body, out_shape=out_shape)(...)


# baseline (device time: 87909 ns/iter reference)
import jax
import jax.numpy as jnp
from jax import lax
from jax.experimental import pallas as pl
from jax.experimental.pallas import tpu as pltpu

N_DEV = 4
N_HOPS = N_DEV - 1


def kernel(x):
    m, n = x.shape
    chunk = m // N_DEV

    def body(x_ref, out_ref, acc_ref, recv_ref, send_sems, recv_sems):
        p = lax.axis_index("i")
        left = (p + N_DEV - 1) % N_DEV
        right = (p + 1) % N_DEV

        barrier_sem = pltpu.get_barrier_semaphore()
        for nbr in (left, right):
            pl.semaphore_signal(
                barrier_sem, inc=1,
                device_id=(nbr,), device_id_type=pl.DeviceIdType.MESH,
            )
        pl.semaphore_wait(barrier_sem, 2)

        acc_ref[...] = x_ref[...].astype(jnp.bfloat16)

        for s in range(N_HOPS):
            send_off = ((p + N_DEV - s) % N_DEV) * chunk
            recv_off = ((p + N_DEV - 1 - s) % N_DEV) * chunk
            rdma = pltpu.make_async_remote_copy(
                src_ref=acc_ref.at[pl.ds(send_off, chunk)],
                dst_ref=recv_ref.at[s],
                send_sem=send_sems.at[s],
                recv_sem=recv_sems.at[s],
                device_id=(right,),
                device_id_type=pl.DeviceIdType.MESH,
            )
            rdma.start()
            rdma.wait()
            acc_ref[pl.ds(recv_off, chunk), :] += recv_ref[s]

        own_off = ((p + 1) % N_DEV) * chunk
        out_ref[pl.ds(own_off, chunk), :] = acc_ref[
            pl.ds(own_off, chunk), :
        ].astype(jnp.float32)

        for s in range(N_HOPS):
            off = ((p + 1 + N_DEV - s) % N_DEV) * chunk
            rdma = pltpu.make_async_remote_copy(
                src_ref=acc_ref.at[pl.ds(off, chunk)],
                dst_ref=acc_ref.at[pl.ds(off, chunk)],
                send_sem=send_sems.at[N_HOPS + s],
                recv_sem=recv_sems.at[N_HOPS + s],
                device_id=(right,),
                device_id_type=pl.DeviceIdType.MESH,
            )
            rdma.start()
            rdma.wait()
            got_off = ((p + N_DEV - s) % N_DEV) * chunk
            out_ref[pl.ds(got_off, chunk), :] = acc_ref[
                pl.ds(got_off, chunk), :
            ].astype(jnp.float32)

    return pl.pallas_call(
        body,
        out_shape=jax.ShapeDtypeStruct((m, n), jnp.float32),
        in_specs=[pl.BlockSpec(memory_space=pltpu.VMEM)],
        out_specs=pl.BlockSpec(memory_space=pltpu.VMEM),
        scratch_shapes=[
            pltpu.VMEM((m, n), jnp.bfloat16),
            pltpu.VMEM((N_HOPS, chunk, n), jnp.bfloat16),
            pltpu.SemaphoreType.DMA((2 * N_HOPS,)),
            pltpu.SemaphoreType.DMA((2 * N_HOPS,)),
        ],
        compiler_params=pltpu.CompilerParams(collective_id=0),
    )(x)


# device time: 52874 ns/iter; 1.6626x vs baseline; 1.6626x over previous
import jax
import jax.numpy as jnp
from jax import lax
from jax.experimental import pallas as pl
from jax.experimental.pallas import tpu as pltpu

N_DEV = 4
N_HOPS = N_DEV - 1


def kernel(x):
    m, n = x.shape
    chunk = m // N_DEV
    nh = n // 2

    def body(x_ref, out_ref, acc_ref, rsr_ref, rsl_ref, send_sems, recv_sems):
        p = lax.axis_index("i")
        left = (p + N_DEV - 1) % N_DEV
        right = (p + 1) % N_DEV

        barrier_sem = pltpu.get_barrier_semaphore()
        for nbr in (left, right):
            pl.semaphore_signal(
                barrier_sem, inc=1,
                device_id=(nbr,), device_id_type=pl.DeviceIdType.MESH,
            )
        pl.semaphore_wait(barrier_sem, 2)

        acc_ref[...] = x_ref[...].astype(jnp.bfloat16)

        for s in range(N_HOPS):
            r_send = ((p + N_DEV - s) % N_DEV) * chunk
            r_recv = ((p + N_DEV - 1 - s) % N_DEV) * chunk
            l_send = ((p + s) % N_DEV) * chunk
            l_recv = ((p + s + 1) % N_DEV) * chunk
            r_rdma = pltpu.make_async_remote_copy(
                src_ref=acc_ref.at[pl.ds(r_send, chunk), pl.ds(0, nh)],
                dst_ref=rsr_ref.at[s],
                send_sem=send_sems.at[s],
                recv_sem=recv_sems.at[s],
                device_id=(right,),
                device_id_type=pl.DeviceIdType.MESH,
            )
            l_rdma = pltpu.make_async_remote_copy(
                src_ref=acc_ref.at[pl.ds(l_send, chunk), pl.ds(nh, nh)],
                dst_ref=rsl_ref.at[s],
                send_sem=send_sems.at[N_HOPS + s],
                recv_sem=recv_sems.at[N_HOPS + s],
                device_id=(left,),
                device_id_type=pl.DeviceIdType.MESH,
            )
            r_rdma.start()
            l_rdma.start()
            r_rdma.wait()
            l_rdma.wait()
            acc_ref[pl.ds(r_recv, chunk), pl.ds(0, nh)] += rsr_ref[s]
            acc_ref[pl.ds(l_recv, chunk), pl.ds(nh, nh)] += rsl_ref[s]

        own_r = ((p + 1) % N_DEV) * chunk
        own_l = ((p + N_DEV - 1) % N_DEV) * chunk
        out_ref[pl.ds(own_r, chunk), pl.ds(0, nh)] = acc_ref[
            pl.ds(own_r, chunk), pl.ds(0, nh)
        ]
        out_ref[pl.ds(own_l, chunk), pl.ds(nh, nh)] = acc_ref[
            pl.ds(own_l, chunk), pl.ds(nh, nh)
        ]

        for s in range(N_HOPS):
            r_off = ((p + 1 + N_DEV - s) % N_DEV) * chunk
            l_off = ((p + N_DEV - 1 + s) % N_DEV) * chunk
            r_rdma = pltpu.make_async_remote_copy(
                src_ref=out_ref.at[pl.ds(r_off, chunk), pl.ds(0, nh)],
                dst_ref=out_ref.at[pl.ds(r_off, chunk), pl.ds(0, nh)],
                send_sem=send_sems.at[2 * N_HOPS + s],
                recv_sem=recv_sems.at[2 * N_HOPS + s],
                device_id=(right,),
                device_id_type=pl.DeviceIdType.MESH,
            )
            l_rdma = pltpu.make_async_remote_copy(
                src_ref=out_ref.at[pl.ds(l_off, chunk), pl.ds(nh, nh)],
                dst_ref=out_ref.at[pl.ds(l_off, chunk), pl.ds(nh, nh)],
                send_sem=send_sems.at[3 * N_HOPS + s],
                recv_sem=recv_sems.at[3 * N_HOPS + s],
                device_id=(left,),
                device_id_type=pl.DeviceIdType.MESH,
            )
            r_rdma.start()
            l_rdma.start()
            r_rdma.wait()
            l_rdma.wait()

    return pl.pallas_call(
        body,
        out_shape=jax.ShapeDtypeStruct((m, n), jnp.bfloat16),
        in_specs=[pl.BlockSpec(memory_space=pltpu.VMEM)],
        out_specs=pl.BlockSpec(memory_space=pltpu.VMEM),
        scratch_shapes=[
            pltpu.VMEM((m, n), jnp.bfloat16),
            pltpu.VMEM((N_HOPS, chunk, nh), jnp.bfloat16),
            pltpu.VMEM((N_HOPS, chunk, nh), jnp.bfloat16),
            pltpu.SemaphoreType.DMA((4 * N_HOPS,)),
            pltpu.SemaphoreType.DMA((4 * N_HOPS,)),
        ],
        compiler_params=pltpu.CompilerParams(collective_id=0),
    )(x)


# device time: 45795 ns/iter; 1.9196x vs baseline; 1.1546x over previous
import jax
import jax.numpy as jnp
from jax import lax
from jax.experimental import pallas as pl
from jax.experimental.pallas import tpu as pltpu

N_DEV = 4
N_HOPS = N_DEV - 1
N_STREAMS = 2


def kernel(x):
    m, n = x.shape
    chunk = m // N_DEV
    sub = chunk // N_STREAMS
    nh = n // 2

    def body(x_ref, out_ref, acc_ref, rsr_ref, rsl_ref, send_sems, recv_sems):
        p = lax.axis_index("i")
        left = (p + N_DEV - 1) % N_DEV
        right = (p + 1) % N_DEV

        def sem(phase, d, k, s):
            return ((phase * 2 + d) * N_STREAMS + k) * N_HOPS + s

        def rs_send_off(d, s):
            c = (p + N_DEV - s) % N_DEV if d == 0 else (p + s) % N_DEV
            return c * chunk

        def rs_recv_off(d, s):
            c = (p + N_DEV - 1 - s) % N_DEV if d == 0 else (p + s + 1) % N_DEV
            return c * chunk

        def own_off(d):
            c = (p + 1) % N_DEV if d == 0 else (p + N_DEV - 1) % N_DEV
            return c * chunk

        def ag_off(d, s):
            c = (p + 1 + N_DEV - s) % N_DEV if d == 0 else (p + N_DEV - 1 + s) % N_DEV
            return c * chunk

        def col(d):
            return pl.ds(0, nh) if d == 0 else pl.ds(nh, nh)

        def nbr(d):
            return (right,) if d == 0 else (left,)

        rs_recv_ref = (rsr_ref, rsl_ref)
        all_rdmas = []

        def start_rs(d, k, s):
            rdma = pltpu.make_async_remote_copy(
                src_ref=acc_ref.at[pl.ds(rs_send_off(d, s) + k * sub, sub), col(d)],
                dst_ref=rs_recv_ref[d].at[s, pl.ds(k * sub, sub), :],
                send_sem=send_sems.at[sem(0, d, k, s)],
                recv_sem=recv_sems.at[sem(0, d, k, s)],
                device_id=nbr(d),
                device_id_type=pl.DeviceIdType.MESH,
            )
            rdma.start()
            all_rdmas.append(rdma)
            return rdma

        def start_ag(d, k, s):
            rows = pl.ds(ag_off(d, s) + k * sub, sub)
            src = acc_ref if s == 0 else out_ref
            rdma = pltpu.make_async_remote_copy(
                src_ref=src.at[rows, col(d)],
                dst_ref=out_ref.at[rows, col(d)],
                send_sem=send_sems.at[sem(1, d, k, s)],
                recv_sem=recv_sems.at[sem(1, d, k, s)],
                device_id=nbr(d),
                device_id_type=pl.DeviceIdType.MESH,
            )
            rdma.start()
            all_rdmas.append(rdma)
            return rdma

        barrier_sem = pltpu.get_barrier_semaphore()
        for d in range(2):
            pl.semaphore_signal(
                barrier_sem, inc=1,
                device_id=nbr(d), device_id_type=pl.DeviceIdType.MESH,
            )
        pl.semaphore_wait(barrier_sem, 2)

        acc_ref[pl.ds(p * chunk, chunk), :] = x_ref[
            pl.ds(p * chunk, chunk), :
        ].astype(jnp.bfloat16)
        rs_rdmas = {}
        for d in range(2):
            for k in range(N_STREAMS):
                rs_rdmas[(d, k, 0)] = start_rs(d, k, 0)
        for j in range(1, N_DEV):
            off = ((p + j) % N_DEV) * chunk
            acc_ref[pl.ds(off, chunk), :] = x_ref[pl.ds(off, chunk), :].astype(
                jnp.bfloat16
            )

        ag_rdmas = {}
        for s in range(N_HOPS):
            for d in range(2):
                for k in range(N_STREAMS):
                    rs_rdmas[(d, k, s)].wait_recv()
                    rows = pl.ds(rs_recv_off(d, s) + k * sub, sub)
                    acc_ref[rows, col(d)] += rs_recv_ref[d][
                        s, pl.ds(k * sub, sub), :
                    ]
                    if s < N_HOPS - 1:
                        rs_rdmas[(d, k, s + 1)] = start_rs(d, k, s + 1)
                    else:
                        ag_rdmas[(d, k, 0)] = start_ag(d, k, 0)
                        own_rows = pl.ds(own_off(d) + k * sub, sub)
                        out_ref[own_rows, col(d)] = acc_ref[own_rows, col(d)]

        for s in range(N_HOPS):
            for d in range(2):
                for k in range(N_STREAMS):
                    ag_rdmas[(d, k, s)].wait_recv()
                    if s < N_HOPS - 1:
                        ag_rdmas[(d, k, s + 1)] = start_ag(d, k, s + 1)

        for rdma in all_rdmas:
            rdma.wait_send()

    return pl.pallas_call(
        body,
        out_shape=jax.ShapeDtypeStruct((m, n), jnp.bfloat16),
        in_specs=[pl.BlockSpec(memory_space=pltpu.VMEM)],
        out_specs=pl.BlockSpec(memory_space=pltpu.VMEM),
        scratch_shapes=[
            pltpu.VMEM((m, n), jnp.bfloat16),
            pltpu.VMEM((N_HOPS, chunk, nh), jnp.bfloat16),
            pltpu.VMEM((N_HOPS, chunk, nh), jnp.bfloat16),
            pltpu.SemaphoreType.DMA((2 * 2 * N_STREAMS * N_HOPS,)),
            pltpu.SemaphoreType.DMA((2 * 2 * N_STREAMS * N_HOPS,)),
        ],
        compiler_params=pltpu.CompilerParams(collective_id=0),
    )(x)


# device time: 44280 ns/iter; 1.9853x vs baseline; 1.0342x over previous
import jax
import jax.numpy as jnp
from jax import lax
from jax.experimental import pallas as pl
from jax.experimental.pallas import tpu as pltpu

N_DEV = 4
N_HOPS = N_DEV - 1
N_STREAMS = 4


def kernel(x):
    m, n = x.shape
    chunk = m // N_DEV
    sub = chunk // N_STREAMS
    nh = n // 2

    def body(x_ref, out_ref, acc_ref, rsr_ref, rsl_ref, send_sems, recv_sems):
        p = lax.axis_index("i")
        left = (p + N_DEV - 1) % N_DEV
        right = (p + 1) % N_DEV

        def sem(phase, d, k, s):
            return ((phase * 2 + d) * N_STREAMS + k) * N_HOPS + s

        def rs_send_off(d, s):
            c = (p + N_DEV - s) % N_DEV if d == 0 else (p + s) % N_DEV
            return c * chunk

        def rs_recv_off(d, s):
            c = (p + N_DEV - 1 - s) % N_DEV if d == 0 else (p + s + 1) % N_DEV
            return c * chunk

        def own_off(d):
            c = (p + 1) % N_DEV if d == 0 else (p + N_DEV - 1) % N_DEV
            return c * chunk

        def ag_off(d, s):
            c = (p + 1 + N_DEV - s) % N_DEV if d == 0 else (p + N_DEV - 1 + s) % N_DEV
            return c * chunk

        def col(d):
            return pl.ds(0, nh) if d == 0 else pl.ds(nh, nh)

        def nbr(d):
            return (right,) if d == 0 else (left,)

        rs_recv_ref = (rsr_ref, rsl_ref)
        all_rdmas = []

        def start_rs(d, k, s):
            rdma = pltpu.make_async_remote_copy(
                src_ref=acc_ref.at[pl.ds(rs_send_off(d, s) + k * sub, sub), col(d)],
                dst_ref=rs_recv_ref[d].at[s, pl.ds(k * sub, sub), :],
                send_sem=send_sems.at[sem(0, d, k, s)],
                recv_sem=recv_sems.at[sem(0, d, k, s)],
                device_id=nbr(d),
                device_id_type=pl.DeviceIdType.MESH,
            )
            rdma.start()
            all_rdmas.append(rdma)
            return rdma

        def start_ag(d, k, s):
            rows = pl.ds(ag_off(d, s) + k * sub, sub)
            src = acc_ref if s == 0 else out_ref
            rdma = pltpu.make_async_remote_copy(
                src_ref=src.at[rows, col(d)],
                dst_ref=out_ref.at[rows, col(d)],
                send_sem=send_sems.at[sem(1, d, k, s)],
                recv_sem=recv_sems.at[sem(1, d, k, s)],
                device_id=nbr(d),
                device_id_type=pl.DeviceIdType.MESH,
            )
            rdma.start()
            all_rdmas.append(rdma)
            return rdma

        barrier_sem = pltpu.get_barrier_semaphore()
        for d in range(2):
            pl.semaphore_signal(
                barrier_sem, inc=1,
                device_id=nbr(d), device_id_type=pl.DeviceIdType.MESH,
            )
        pl.semaphore_wait(barrier_sem, 2)

        acc_ref[pl.ds(p * chunk, chunk), :] = x_ref[
            pl.ds(p * chunk, chunk), :
        ].astype(jnp.bfloat16)
        rs_rdmas = {}
        for d in range(2):
            for k in range(N_STREAMS):
                rs_rdmas[(d, k, 0)] = start_rs(d, k, 0)
        for j in range(1, N_DEV):
            off = ((p + j) % N_DEV) * chunk
            acc_ref[pl.ds(off, chunk), :] = x_ref[pl.ds(off, chunk), :].astype(
                jnp.bfloat16
            )

        ag_rdmas = {}
        for s in range(N_HOPS):
            for d in range(2):
                for k in range(N_STREAMS):
                    rs_rdmas[(d, k, s)].wait_recv()
                    rows = pl.ds(rs_recv_off(d, s) + k * sub, sub)
                    acc_ref[rows, col(d)] += rs_recv_ref[d][
                        s, pl.ds(k * sub, sub), :
                    ]
                    if s < N_HOPS - 1:
                        rs_rdmas[(d, k, s + 1)] = start_rs(d, k, s + 1)
                    else:
                        ag_rdmas[(d, k, 0)] = start_ag(d, k, 0)
                        own_rows = pl.ds(own_off(d) + k * sub, sub)
                        out_ref[own_rows, col(d)] = acc_ref[own_rows, col(d)]

        for s in range(N_HOPS):
            for d in range(2):
                for k in range(N_STREAMS):
                    ag_rdmas[(d, k, s)].wait_recv()
                    if s < N_HOPS - 1:
                        ag_rdmas[(d, k, s + 1)] = start_ag(d, k, s + 1)

        for rdma in all_rdmas:
            rdma.wait_send()

    return pl.pallas_call(
        body,
        out_shape=jax.ShapeDtypeStruct((m, n), jnp.bfloat16),
        in_specs=[pl.BlockSpec(memory_space=pltpu.VMEM)],
        out_specs=pl.BlockSpec(memory_space=pltpu.VMEM),
        scratch_shapes=[
            pltpu.VMEM((m, n), jnp.bfloat16),
            pltpu.VMEM((N_HOPS, chunk, nh), jnp.bfloat16),
            pltpu.VMEM((N_HOPS, chunk, nh), jnp.bfloat16),
            pltpu.SemaphoreType.DMA((2 * 2 * N_STREAMS * N_HOPS,)),
            pltpu.SemaphoreType.DMA((2 * 2 * N_STREAMS * N_HOPS,)),
        ],
        compiler_params=pltpu.CompilerParams(collective_id=0),
    )(x)
